# baseline (speedup 1.0000x reference)
import functools

import jax
import jax.numpy as jnp
from jax import lax
from jax.experimental import pallas as pl
from jax.experimental.pallas import tpu as pltpu
from jax.experimental.pallas import tpu_sc as plsc

_NUM_CORES = 2
_NUM_SUBCORES = 16
_NUM_WORKERS = _NUM_CORES * _NUM_SUBCORES
_BUF_ROWS = 40
_NBUF = 3


@functools.lru_cache(maxsize=None)
def _broadcast_kernel(batch, seq, hidden):
    rows_per_worker = seq // _NUM_WORKERS
    chunk_rows = []
    r = rows_per_worker
    while r > 0:
        c = min(r, _BUF_ROWS)
        chunk_rows.append(c)
        r -= c
    chunk_offs = [sum(chunk_rows[:i]) for i in range(len(chunk_rows))]
    num_chunks = len(chunk_rows)
    mesh = plsc.VectorSubcoreMesh(core_axis_name="c", subcore_axis_name="s")

    @functools.partial(
        pl.kernel,
        mesh=mesh,
        out_type=jax.ShapeDtypeStruct((batch, seq, hidden), jnp.float32),
        scratch_types=(
            [pltpu.VMEM((_BUF_ROWS, hidden), jnp.float32)] * _NBUF
            + [pltpu.SemaphoreType.DMA] * (1 + _NBUF)
        ),
    )
    def k(table_hbm, out_hbm, *rest):
        bufs = rest[:_NBUF]
        ld = rest[_NBUF]
        sts = rest[_NBUF + 1:]
        wid = lax.axis_index("s") * _NUM_CORES + lax.axis_index("c")
        base = wid * rows_per_worker
        loads = [None] * num_chunks
        stores = [None] * num_chunks

        def start_load(i):
            n = chunk_rows[i]
            return pltpu.async_copy(
                table_hbm.at[pl.ds(base + chunk_offs[i], n), :],
                bufs[i % _NBUF].at[pl.ds(0, n), :], ld)

        for i in range(min(_NBUF, num_chunks)):
            loads[i] = start_load(i)
        drained = 0
        for i in range(num_chunks):
            n = chunk_rows[i]
            loads[i].wait()
            row0 = base + chunk_offs[i]
            stores[i] = [
                pltpu.async_copy(
                    bufs[i % _NBUF].at[pl.ds(0, n), :],
                    out_hbm.at[b, pl.ds(row0, n), :], sts[i % _NBUF])
                for b in range(batch)
            ]
            nxt = i - 1 + _NBUF
            if i >= 1 and nxt < num_chunks:
                for h in stores[i - 1]:
                    h.wait()
                drained = i
                loads[nxt] = start_load(nxt)
        for i in range(drained, num_chunks):
            for h in stores[i]:
                h.wait()

    return k


def kernel(x, pos_table):
    batch = x.shape[0]
    seq, hidden = pos_table.shape
    return _broadcast_kernel(batch, seq, hidden)(pos_table)


# speedup vs baseline: 1.0077x; 1.0077x over previous
import functools

import jax
import jax.numpy as jnp
from jax import lax
from jax.experimental import pallas as pl
from jax.experimental.pallas import tpu as pltpu
from jax.experimental.pallas import tpu_sc as plsc

_NUM_CORES = 2
_NUM_SUBCORES = 16
_NUM_WORKERS = _NUM_CORES * _NUM_SUBCORES
_BUF_ROWS = 56
_NBUF = 2


@functools.lru_cache(maxsize=None)
def _broadcast_kernel(batch, seq, hidden):
    rows_per_worker = seq // _NUM_WORKERS
    chunk_rows = []
    r = rows_per_worker
    while r > 0:
        c = min(r, _BUF_ROWS)
        chunk_rows.append(c)
        r -= c
    chunk_offs = [sum(chunk_rows[:i]) for i in range(len(chunk_rows))]
    num_chunks = len(chunk_rows)
    mesh = plsc.VectorSubcoreMesh(core_axis_name="c", subcore_axis_name="s")

    @functools.partial(
        pl.kernel,
        mesh=mesh,
        out_type=jax.ShapeDtypeStruct((batch, seq, hidden), jnp.float32),
        scratch_types=(
            [pltpu.VMEM((_BUF_ROWS, hidden), jnp.float32)] * _NBUF
            + [pltpu.SemaphoreType.DMA] * (1 + _NBUF)
        ),
    )
    def k(table_hbm, out_hbm, *rest):
        bufs = rest[:_NBUF]
        ld = rest[_NBUF]
        sts = rest[_NBUF + 1:]
        wid = lax.axis_index("s") * _NUM_CORES + lax.axis_index("c")
        base = wid * rows_per_worker
        loads = [None] * num_chunks
        stores = [None] * num_chunks

        def start_load(i):
            n = chunk_rows[i]
            return pltpu.async_copy(
                table_hbm.at[pl.ds(base + chunk_offs[i], n), :],
                bufs[i % _NBUF].at[pl.ds(0, n), :], ld)

        for i in range(min(_NBUF, num_chunks)):
            loads[i] = start_load(i)
        drained = 0
        for i in range(num_chunks):
            n = chunk_rows[i]
            loads[i].wait()
            row0 = base + chunk_offs[i]
            stores[i] = [
                pltpu.async_copy(
                    bufs[i % _NBUF].at[pl.ds(0, n), :],
                    out_hbm.at[b, pl.ds(row0, n), :], sts[i % _NBUF])
                for b in range(batch)
            ]
            nxt = i - 1 + _NBUF
            if i >= 1 and nxt < num_chunks:
                for h in stores[i - 1]:
                    h.wait()
                drained = i
                loads[nxt] = start_load(nxt)
        for i in range(drained, num_chunks):
            for h in stores[i]:
                h.wait()

    return k


def kernel(x, pos_table):
    batch = x.shape[0]
    seq, hidden = pos_table.shape
    return _broadcast_kernel(batch, seq, hidden)(pos_table)


# speedup vs baseline: 1.0101x; 1.0024x over previous
import functools

import jax
import jax.numpy as jnp
from jax import lax
from jax.experimental import pallas as pl
from jax.experimental.pallas import tpu as pltpu
from jax.experimental.pallas import tpu_sc as plsc

_NUM_CORES = 2
_NUM_SUBCORES = 16
_NUM_WORKERS = _NUM_CORES * _NUM_SUBCORES
_BUF_ROWS = 56
_NBUF = 2


@functools.lru_cache(maxsize=None)
def _broadcast_kernel(batch, seq, hidden):
    rows_per_worker = seq // _NUM_WORKERS
    chunk_rows = []
    r = rows_per_worker
    while r > 0:
        c = min(r, _BUF_ROWS)
        chunk_rows.append(c)
        r -= c
    chunk_offs = [sum(chunk_rows[:i]) for i in range(len(chunk_rows))]
    num_chunks = len(chunk_rows)
    mesh = plsc.VectorSubcoreMesh(core_axis_name="c", subcore_axis_name="s")

    @functools.partial(
        pl.kernel,
        mesh=mesh,
        out_type=jax.ShapeDtypeStruct((batch, seq, hidden), jnp.float32),
        scratch_types=(
            [pltpu.VMEM((_BUF_ROWS, hidden), jnp.float32)] * _NBUF
            + [pltpu.SemaphoreType.DMA] * (1 + _NBUF)
        ),
    )
    def k(table_hbm, out_hbm, *rest):
        bufs = rest[:_NBUF]
        ld = rest[_NBUF]
        sts = rest[_NBUF + 1:]
        wid = lax.axis_index("s") * _NUM_CORES + lax.axis_index("c")
        base = wid * rows_per_worker
        loads = [None] * num_chunks
        stores = [None] * num_chunks

        def start_load(i):
            n = chunk_rows[i]
            return pltpu.async_copy(
                table_hbm.at[pl.ds(base + chunk_offs[i], n), :],
                bufs[i % _NBUF].at[pl.ds(0, n), :], ld)

        for i in range(min(_NBUF, num_chunks)):
            loads[i] = start_load(i)
        drained = 0
        for i in range(num_chunks):
            n = chunk_rows[i]
            loads[i].wait()
            row0 = base + chunk_offs[i]
            stores[i] = [
                pltpu.async_copy(
                    bufs[i % _NBUF].at[pl.ds(0, n), :],
                    out_hbm.at[(wid + b) % batch, pl.ds(row0, n), :],
                    sts[i % _NBUF])
                for b in range(batch)
            ]
            nxt = i - 1 + _NBUF
            if i >= 1 and nxt < num_chunks:
                for h in stores[i - 1]:
                    h.wait()
                drained = i
                loads[nxt] = start_load(nxt)
        for i in range(drained, num_chunks):
            for h in stores[i]:
                h.wait()

    return k


def kernel(x, pos_table):
    batch = x.shape[0]
    seq, hidden = pos_table.shape
    return _broadcast_kernel(batch, seq, hidden)(pos_table)
